# baseline (device time: 83691 ns/iter reference)
import jax
import jax.numpy as jnp
from jax import lax
from jax.experimental import pallas as pl
from jax.experimental.pallas import tpu as pltpu

N_Z = 4
N_HOP = N_Z - 1


def kernel(x):
    m_per, n = x.shape
    qr = m_per // 4
    hr = qr // 2

    def body(x_ref, out_ref, stage,
             ring_s, ring_r, sx_s, sx_r, sy_s, sy_r,
             rly_s, rly_r, rlx_s, rlx_r, cp_sem):
        my_x = lax.axis_index("x")
        my_y = lax.axis_index("y")
        my_z = lax.axis_index("z")
        ox = 1 - my_x
        oy = 1 - my_y
        zr = jnp.mod(my_z + 1, N_Z)
        zl = jnp.mod(my_z - 1, N_Z)
        q = 2 * my_x + my_y
        qx = 2 * ox + my_y
        qy = 2 * my_x + oy

        barrier_sem = pltpu.get_barrier_semaphore()
        for dev in ((my_x, my_y, zl), (my_x, my_y, zr),
                    (ox, my_y, my_z), (my_x, oy, my_z)):
            pl.semaphore_signal(barrier_sem, inc=1, device_id=dev,
                                device_id_type=pl.DeviceIdType.MESH)
        pl.semaphore_wait(barrier_sem, 4)

        def quarter(c, qi):
            return stage.at[pl.ds(c * m_per + qi * qr, qr), :]

        all_rdmas = []
        copies = []

        def copy_out(start_row, nrows):
            cp = pltpu.make_async_copy(
                stage.at[pl.ds(start_row, nrows), :],
                out_ref.at[pl.ds(start_row, nrows), :],
                cp_sem.at[len(copies)])
            copies.append(cp)
            cp.start()

        def ring_hop(h):
            cs = jnp.mod(my_z - h, N_Z)
            src = (x_ref.at[pl.ds(q * qr, qr), :] if h == 0
                   else quarter(cs, q))
            r = pltpu.make_async_remote_copy(
                src_ref=src, dst_ref=quarter(cs, q),
                send_sem=ring_s.at[h], recv_sem=ring_r.at[h],
                device_id=(my_x, my_y, zr),
                device_id_type=pl.DeviceIdType.MESH)
            all_rdmas.append(r)
            return r

        ring = [None] * N_HOP
        ring[0] = ring_hop(0)
        ring[0].start()

        own = pltpu.make_async_copy(
            x_ref, out_ref.at[pl.ds(my_z * m_per, m_per), :],
            cp_sem.at[15])
        own.start()

        s4 = [None] * N_HOP
        s5 = [None] * N_HOP
        for h in range(N_HOP):
            cr = jnp.mod(my_z - 1 - h, N_Z)
            ring[h].wait_recv()
            copy_out(cr * m_per + q * qr, qr)
            if h + 1 < N_HOP:
                ring[h + 1] = ring_hop(h + 1)
                ring[h + 1].start()
            sx = pltpu.make_async_remote_copy(
                src_ref=quarter(cr, q), dst_ref=quarter(cr, q),
                send_sem=sx_s.at[h], recv_sem=sx_r.at[h],
                device_id=(ox, my_y, my_z),
                device_id_type=pl.DeviceIdType.MESH)
            sy = pltpu.make_async_remote_copy(
                src_ref=quarter(cr, q), dst_ref=quarter(cr, q),
                send_sem=sy_s.at[h], recv_sem=sy_r.at[h],
                device_id=(my_x, oy, my_z),
                device_id_type=pl.DeviceIdType.MESH)
            all_rdmas += [sx, sy]
            sx.start()
            sy.start()
            sx.wait_recv()
            copy_out(cr * m_per + qx * qr, qr)
            s4[h] = pltpu.make_async_remote_copy(
                src_ref=stage.at[pl.ds(cr * m_per + qx * qr + hr, hr), :],
                dst_ref=stage.at[pl.ds(cr * m_per + qx * qr + hr, hr), :],
                send_sem=rly_s.at[h], recv_sem=rly_r.at[h],
                device_id=(my_x, oy, my_z),
                device_id_type=pl.DeviceIdType.MESH)
            all_rdmas.append(s4[h])
            s4[h].start()
            sy.wait_recv()
            copy_out(cr * m_per + qy * qr, qr)
            s5[h] = pltpu.make_async_remote_copy(
                src_ref=stage.at[pl.ds(cr * m_per + qy * qr, hr), :],
                dst_ref=stage.at[pl.ds(cr * m_per + qy * qr, hr), :],
                send_sem=rlx_s.at[h], recv_sem=rlx_r.at[h],
                device_id=(ox, my_y, my_z),
                device_id_type=pl.DeviceIdType.MESH)
            all_rdmas.append(s5[h])
            s5[h].start()

        qd = 2 * ox + oy
        for h in range(N_HOP):
            cr = jnp.mod(my_z - 1 - h, N_Z)
            s5[h].wait_recv()
            copy_out(cr * m_per + qd * qr, hr)
            s4[h].wait_recv()
            copy_out(cr * m_per + qd * qr + hr, hr)

        for r in all_rdmas:
            r.wait_send()
        for cp in copies:
            cp.wait()
        own.wait()

    out_shape = jax.ShapeDtypeStruct((N_Z * m_per, n), x.dtype)
    sem = pltpu.SemaphoreType.DMA((N_HOP,))
    return pl.pallas_call(
        body,
        out_shape=out_shape,
        in_specs=[pl.BlockSpec(memory_space=pltpu.VMEM)],
        out_specs=pl.BlockSpec(memory_space=pltpu.MemorySpace.HBM),
        scratch_shapes=[pltpu.VMEM((N_Z * m_per, n), x.dtype)]
        + [sem] * 10
        + [pltpu.SemaphoreType.DMA((16,))],
        compiler_params=pltpu.CompilerParams(collective_id=0),
    )(x)


# device time: 75140 ns/iter; 1.1138x vs baseline; 1.1138x over previous
import jax
import jax.numpy as jnp
from jax import lax
from jax.experimental import pallas as pl
from jax.experimental.pallas import tpu as pltpu

N_Z = 4
N_HOP = N_Z - 1
FA, FB, XA, XB, YA, YB, RY, RX = range(8)


def kernel(x):
    m_per, n = x.shape
    qr = m_per // 4
    hr = qr // 2

    def body(x_ref, out_ref, stage, send_s, recv_s, cp_sem):
        my_x = lax.axis_index("x")
        my_y = lax.axis_index("y")
        my_z = lax.axis_index("z")
        ox = 1 - my_x
        oy = 1 - my_y
        zr = jnp.mod(my_z + 1, N_Z)
        zl = jnp.mod(my_z - 1, N_Z)
        q = 2 * my_x + my_y
        qx = 2 * ox + my_y
        qy = 2 * my_x + oy
        qd = 2 * ox + oy

        barrier_sem = pltpu.get_barrier_semaphore()
        for dev in ((my_x, my_y, zl), (my_x, my_y, zr),
                    (ox, my_y, my_z), (my_x, oy, my_z)):
            pl.semaphore_signal(barrier_sem, inc=1, device_id=dev,
                                device_id_type=pl.DeviceIdType.MESH)
        pl.semaphore_wait(barrier_sem, 4)

        def half(c, qi, piece):
            return stage.at[pl.ds(c * m_per + qi * qr + piece * hr, hr), :]

        all_rdmas = []
        copies = []

        def copy_out(start_row, nrows):
            cp = pltpu.make_async_copy(
                stage.at[pl.ds(start_row, nrows), :],
                out_ref.at[pl.ds(start_row, nrows), :],
                cp_sem.at[len(copies)])
            copies.append(cp)
            cp.start()

        def rdma(flow, h, src, dst, dev, track=True):
            r = pltpu.make_async_remote_copy(
                src_ref=src, dst_ref=dst,
                send_sem=send_s.at[flow, h], recv_sem=recv_s.at[flow, h],
                device_id=dev, device_id_type=pl.DeviceIdType.MESH)
            if track:
                all_rdmas.append(r)
            return r

        def ring_hop(flow, h):
            piece = 0 if flow == FA else 1
            step = -h if flow == FA else h
            cs = jnp.mod(my_z + step, N_Z)
            src = (x_ref.at[pl.ds(q * qr + piece * hr, hr), :] if h == 0
                   else half(cs, q, piece))
            tgt = (my_x, my_y, zr if flow == FA else zl)
            return rdma(flow, h, src, half(cs, q, piece), tgt)

        ringA = [None] * N_HOP
        ringB = [None] * N_HOP
        ringA[0] = ring_hop(FA, 0)
        ringB[0] = ring_hop(FB, 0)
        ringA[0].start()
        ringB[0].start()

        own = pltpu.make_async_copy(
            x_ref, out_ref.at[pl.ds(my_z * m_per, m_per), :],
            cp_sem.at[31])
        own.start()

        s4 = [None] * N_HOP
        s5 = [None] * N_HOP
        for h in range(N_HOP):
            cA = jnp.mod(my_z - 1 - h, N_Z)
            cB = jnp.mod(my_z + 1 + h, N_Z)

            ringA[h].wait_recv()
            copy_out(cA * m_per + q * qr, hr)
            if h + 1 < N_HOP:
                ringA[h + 1] = ring_hop(FA, h + 1)
                ringA[h + 1].start()
            rdma(XA, h, half(cA, q, 0), half(cA, q, 0),
                 (ox, my_y, my_z)).start()
            rdma(YA, h, half(cA, q, 0), half(cA, q, 0),
                 (my_x, oy, my_z)).start()

            ringB[h].wait_recv()
            copy_out(cB * m_per + q * qr + hr, hr)
            if h + 1 < N_HOP:
                ringB[h + 1] = ring_hop(FB, h + 1)
                ringB[h + 1].start()
            sxB = rdma(XB, h, half(cB, q, 1), half(cB, q, 1),
                       (ox, my_y, my_z))
            syB = rdma(YB, h, half(cB, q, 1), half(cB, q, 1),
                       (my_x, oy, my_z))
            sxB.start()
            syB.start()

            syA_in = rdma(YA, h, half(cA, q, 0), half(cA, q, 0),
                          (my_x, oy, my_z), track=False)
            syA_in.wait_recv()
            copy_out(cA * m_per + qy * qr, hr)
            s5[h] = rdma(RX, h, half(cA, qy, 0), half(cA, qy, 0),
                         (ox, my_y, my_z))
            s5[h].start()

            sxB_in = rdma(XB, h, half(cB, q, 1), half(cB, q, 1),
                          (ox, my_y, my_z), track=False)
            sxB_in.wait_recv()
            copy_out(cB * m_per + qx * qr + hr, hr)
            s4[h] = rdma(RY, h, half(cB, qx, 1), half(cB, qx, 1),
                         (my_x, oy, my_z))
            s4[h].start()

            sxA_in = rdma(XA, h, half(cA, q, 0), half(cA, q, 0),
                          (ox, my_y, my_z), track=False)
            sxA_in.wait_recv()
            copy_out(cA * m_per + qx * qr, hr)
            syB_in = rdma(YB, h, half(cB, q, 1), half(cB, q, 1),
                          (my_x, oy, my_z), track=False)
            syB_in.wait_recv()
            copy_out(cB * m_per + qy * qr + hr, hr)

        for h in range(N_HOP):
            cA = jnp.mod(my_z - 1 - h, N_Z)
            cB = jnp.mod(my_z + 1 + h, N_Z)
            s5[h].wait_recv()
            copy_out(cA * m_per + qd * qr, hr)
            s4[h].wait_recv()
            copy_out(cB * m_per + qd * qr + hr, hr)

        for r in all_rdmas:
            r.wait_send()
        for cp in copies:
            cp.wait()
        own.wait()

    out_shape = jax.ShapeDtypeStruct((N_Z * m_per, n), x.dtype)
    return pl.pallas_call(
        body,
        out_shape=out_shape,
        in_specs=[pl.BlockSpec(memory_space=pltpu.VMEM)],
        out_specs=pl.BlockSpec(memory_space=pltpu.MemorySpace.HBM),
        scratch_shapes=[
            pltpu.VMEM((N_Z * m_per, n), x.dtype),
            pltpu.SemaphoreType.DMA((8, N_HOP)),
            pltpu.SemaphoreType.DMA((8, N_HOP)),
            pltpu.SemaphoreType.DMA((32,)),
        ],
        compiler_params=pltpu.CompilerParams(collective_id=0),
    )(x)


# device time: 69576 ns/iter; 1.2029x vs baseline; 1.0800x over previous
import jax
import jax.numpy as jnp
from jax import lax
from jax.experimental import pallas as pl
from jax.experimental.pallas import tpu as pltpu

N_Z = 4
N_HOP = N_Z - 1
FAQ, FAD, FBQ, FBD, XA, XB, YA, YB, RY, RX = range(10)


def kernel(x):
    m_per, n = x.shape
    qr = m_per // 4
    hr = qr // 2
    dr = 88
    rr = hr - dr

    def body(x_ref, out_ref, stage, send_s, recv_s, cp_sem):
        my_x = lax.axis_index("x")
        my_y = lax.axis_index("y")
        my_z = lax.axis_index("z")
        ox = 1 - my_x
        oy = 1 - my_y
        zr = jnp.mod(my_z + 1, N_Z)
        zl = jnp.mod(my_z - 1, N_Z)
        q = 2 * my_x + my_y
        qx = 2 * ox + my_y
        qy = 2 * my_x + oy
        qd = 2 * ox + oy

        barrier_sem = pltpu.get_barrier_semaphore()
        for dev in ((my_x, my_y, zl), (my_x, my_y, zr),
                    (ox, my_y, my_z), (my_x, oy, my_z)):
            pl.semaphore_signal(barrier_sem, inc=1, device_id=dev,
                                device_id_type=pl.DeviceIdType.MESH)
        pl.semaphore_wait(barrier_sem, 4)

        def piece(c, qi, off, ln):
            return stage.at[pl.ds(c * m_per + qi * qr + off, ln), :]

        all_rdmas = []
        copies = []

        def copy_out(start_row, nrows):
            cp = pltpu.make_async_copy(
                stage.at[pl.ds(start_row, nrows), :],
                out_ref.at[pl.ds(start_row, nrows), :],
                cp_sem.at[len(copies)])
            copies.append(cp)
            cp.start()

        def rdma(flow, h, src, dst, dev, track=True):
            r = pltpu.make_async_remote_copy(
                src_ref=src, dst_ref=dst,
                send_sem=send_s.at[flow, h], recv_sem=recv_s.at[flow, h],
                device_id=dev, device_id_type=pl.DeviceIdType.MESH)
            if track:
                all_rdmas.append(r)
            return r

        def ring_hop(flow, h):
            fwd = flow in (FAQ, FAD)
            qi, off, ln = {
                FAQ: (q, 0, hr), FAD: (qd, 0, dr),
                FBQ: (q, hr, hr), FBD: (qd, hr, dr),
            }[flow]
            cs = jnp.mod(my_z + (-h if fwd else h), N_Z)
            src = (x_ref.at[pl.ds(qi * qr + off, ln), :] if h == 0
                   else piece(cs, qi, off, ln))
            tgt = (my_x, my_y, zr if fwd else zl)
            return rdma(flow, h, src, piece(cs, qi, off, ln), tgt)

        rings = {f: [None] * N_HOP for f in (FAQ, FAD, FBQ, FBD)}
        for f in (FAQ, FBQ, FAD, FBD):
            rings[f][0] = ring_hop(f, 0)
            rings[f][0].start()

        own = pltpu.make_async_copy(
            x_ref, out_ref.at[pl.ds(my_z * m_per, m_per), :],
            cp_sem.at[31])
        own.start()

        s4 = [None] * N_HOP
        s5 = [None] * N_HOP
        for h in range(N_HOP):
            cA = jnp.mod(my_z - 1 - h, N_Z)
            cB = jnp.mod(my_z + 1 + h, N_Z)

            def advance(flow, c, qi, off, ln):
                rings[flow][h].wait_recv()
                copy_out(c * m_per + qi * qr + off, ln)
                if h + 1 < N_HOP:
                    rings[flow][h + 1] = ring_hop(flow, h + 1)
                    rings[flow][h + 1].start()

            advance(FAQ, cA, q, 0, hr)
            rdma(XA, h, piece(cA, q, 0, hr), piece(cA, q, 0, hr),
                 (ox, my_y, my_z)).start()
            rdma(YA, h, piece(cA, q, 0, hr), piece(cA, q, 0, hr),
                 (my_x, oy, my_z)).start()
            advance(FBQ, cB, q, hr, hr)
            rdma(XB, h, piece(cB, q, hr, hr), piece(cB, q, hr, hr),
                 (ox, my_y, my_z)).start()
            rdma(YB, h, piece(cB, q, hr, hr), piece(cB, q, hr, hr),
                 (my_x, oy, my_z)).start()
            advance(FAD, cA, qd, 0, dr)
            advance(FBD, cB, qd, hr, dr)

            rdma(YA, h, piece(cA, q, 0, hr), piece(cA, q, 0, hr),
                 (my_x, oy, my_z), track=False).wait_recv()
            copy_out(cA * m_per + qy * qr, hr)
            s5[h] = rdma(RX, h, piece(cA, qy, dr, rr),
                         piece(cA, qy, dr, rr), (ox, my_y, my_z))
            s5[h].start()

            rdma(XB, h, piece(cB, q, hr, hr), piece(cB, q, hr, hr),
                 (ox, my_y, my_z), track=False).wait_recv()
            copy_out(cB * m_per + qx * qr + hr, hr)
            s4[h] = rdma(RY, h, piece(cB, qx, hr + dr, rr),
                         piece(cB, qx, hr + dr, rr), (my_x, oy, my_z))
            s4[h].start()

            rdma(XA, h, piece(cA, q, 0, hr), piece(cA, q, 0, hr),
                 (ox, my_y, my_z), track=False).wait_recv()
            copy_out(cA * m_per + qx * qr, hr)
            rdma(YB, h, piece(cB, q, hr, hr), piece(cB, q, hr, hr),
                 (my_x, oy, my_z), track=False).wait_recv()
            copy_out(cB * m_per + qy * qr + hr, hr)

        for h in range(N_HOP):
            cA = jnp.mod(my_z - 1 - h, N_Z)
            cB = jnp.mod(my_z + 1 + h, N_Z)
            s5[h].wait_recv()
            copy_out(cA * m_per + qd * qr + dr, rr)
            s4[h].wait_recv()
            copy_out(cB * m_per + qd * qr + hr + dr, rr)

        for r in all_rdmas:
            r.wait_send()
        for cp in copies:
            cp.wait()
        own.wait()

    out_shape = jax.ShapeDtypeStruct((N_Z * m_per, n), x.dtype)
    return pl.pallas_call(
        body,
        out_shape=out_shape,
        in_specs=[pl.BlockSpec(memory_space=pltpu.VMEM)],
        out_specs=pl.BlockSpec(memory_space=pltpu.MemorySpace.HBM),
        scratch_shapes=[
            pltpu.VMEM((N_Z * m_per, n), x.dtype),
            pltpu.SemaphoreType.DMA((10, N_HOP)),
            pltpu.SemaphoreType.DMA((10, N_HOP)),
            pltpu.SemaphoreType.DMA((32,)),
        ],
        compiler_params=pltpu.CompilerParams(collective_id=0),
    )(x)
